# baseline (speedup 1.0000x reference)
import functools

import jax
import jax.numpy as jnp
from jax import lax
from jax.experimental import pallas as pl
from jax.experimental.pallas import tpu as pltpu
from jax.experimental.pallas import tpu_sc as plsc

_N = 16384
_C = 1000

_NC = 2
_NS = 16
_NW = _NC * _NS
_RPW = _N // _NW
_CHUNK = 128
_NCH = _RPW // _CHUNK
_TROWS = _N // _CHUNK
_L = 16

_BROWS = 1024
_WROWS = _BROWS // _CHUNK


def _sc_alpha_body(t_hbm, a_hbm, at_hbm, t_v, at_v, sem):
    wid = lax.axis_index("s") * _NC + lax.axis_index("c")
    r0 = wid * _NCH
    pltpu.sync_copy(t_hbm.at[pl.ds(r0, _NCH)], t_v)
    copies = [
        pltpu.async_copy(a_hbm.at[t_v.at[ch]], at_v.at[ch], sem)
        for ch in range(_NCH)
    ]
    for cp in copies:
        cp.wait()
    pltpu.sync_copy(at_v, at_hbm.at[pl.ds(r0, _NCH)])


@functools.cache
def _sc_alpha():
    return functools.partial(
        pl.kernel,
        mesh=plsc.VectorSubcoreMesh(core_axis_name="c", subcore_axis_name="s"),
        out_type=jax.ShapeDtypeStruct((_TROWS, _CHUNK), jnp.float32),
        scratch_types=[
            pltpu.VMEM((_NCH, _CHUNK), jnp.int32),
            pltpu.VMEM((_NCH, _CHUNK), jnp.float32),
            pltpu.SemaphoreType.DMA,
        ],
    )(_sc_alpha_body)


def _tc_dense_body(x_ref, t_ref, w_ref):
    x = x_ref[...]
    cols = lax.broadcasted_iota(jnp.int32, (_BROWS, _C), 1)
    onehot = (cols == t_ref[...][:, None]).astype(jnp.float32)
    xt = jnp.sum(x * onehot, axis=1)
    m = jnp.max(x, axis=1)
    s = jnp.sum(jnp.exp(x - m[:, None]), axis=1)
    logp = xt - m - jnp.log(s)
    p = jnp.exp(logp)
    q = 1.0 - p
    w_ref[...] = (q * q * logp).reshape(_WROWS, _CHUNK)


def _tc_dense(x, t):
    return pl.pallas_call(
        _tc_dense_body,
        grid=(_N // _BROWS,),
        in_specs=[
            pl.BlockSpec((_BROWS, _C), lambda i: (i, 0)),
            pl.BlockSpec((_BROWS,), lambda i: (i,)),
        ],
        out_specs=pl.BlockSpec((_WROWS, _CHUNK), lambda i: (i, 0)),
        out_shape=jax.ShapeDtypeStruct((_TROWS, _CHUNK), jnp.float32),
        compiler_params=pltpu.CompilerParams(
            dimension_semantics=("parallel",)),
    )(x, t)


def _tc_combine_body(at_ref, w_ref, o_ref):
    o_ref[0, 0] = -jnp.sum(at_ref[...] * w_ref[...]) * (1.0 / _N)


def _tc_combine(at, w):
    return pl.pallas_call(
        _tc_combine_body,
        out_specs=pl.BlockSpec(memory_space=pltpu.SMEM),
        out_shape=jax.ShapeDtypeStruct((1, 1), jnp.float32),
    )(at, w)


def kernel(inputs, targets, alpha, device=0):
    t = targets.astype(jnp.int32)
    a_flat = alpha.reshape(-1).astype(jnp.float32)
    at = _sc_alpha()(t.reshape(_TROWS, _CHUNK), a_flat)
    w = _tc_dense(inputs, t)
    loss = _tc_combine(at, w)
    return loss[0, 0]


# speedup vs baseline: 1.0187x; 1.0187x over previous
import functools

import jax
import jax.numpy as jnp
from jax import lax
from jax.experimental import pallas as pl
from jax.experimental.pallas import tpu as pltpu
from jax.experimental.pallas import tpu_sc as plsc

_N = 16384
_C = 1000

_NC = 2
_NS = 16
_NW = _NC * _NS
_RPW = _N // _NW
_CHUNK = 128
_NCH = _RPW // _CHUNK
_TROWS = _N // _CHUNK
_L = 16

_BROWS = 4096
_WROWS = _BROWS // _CHUNK


def _sc_alpha_body(t_hbm, a_hbm, at_hbm, t_v, at_v, sem):
    wid = lax.axis_index("s") * _NC + lax.axis_index("c")
    r0 = wid * _NCH
    pltpu.sync_copy(t_hbm.at[pl.ds(r0, _NCH)], t_v)
    copies = [
        pltpu.async_copy(a_hbm.at[t_v.at[ch]], at_v.at[ch], sem)
        for ch in range(_NCH)
    ]
    for cp in copies:
        cp.wait()
    pltpu.sync_copy(at_v, at_hbm.at[pl.ds(r0, _NCH)])


@functools.cache
def _sc_alpha():
    return functools.partial(
        pl.kernel,
        mesh=plsc.VectorSubcoreMesh(core_axis_name="c", subcore_axis_name="s"),
        out_type=jax.ShapeDtypeStruct((_TROWS, _CHUNK), jnp.float32),
        scratch_types=[
            pltpu.VMEM((_NCH, _CHUNK), jnp.int32),
            pltpu.VMEM((_NCH, _CHUNK), jnp.float32),
            pltpu.SemaphoreType.DMA,
        ],
    )(_sc_alpha_body)


def _tc_dense_body(x_ref, t_ref, w_ref):
    x = x_ref[...]
    cols = lax.broadcasted_iota(jnp.int32, (_BROWS, _C), 1)
    onehot = (cols == t_ref[...][:, None]).astype(jnp.float32)
    xt = jnp.sum(x * onehot, axis=1)
    m = jnp.max(x, axis=1)
    s = jnp.sum(jnp.exp(x - m[:, None]), axis=1)
    logp = xt - m - jnp.log(s)
    p = jnp.exp(logp)
    q = 1.0 - p
    w_ref[...] = (q * q * logp).reshape(_WROWS, _CHUNK)


def _tc_dense(x, t):
    return pl.pallas_call(
        _tc_dense_body,
        grid=(_N // _BROWS,),
        in_specs=[
            pl.BlockSpec((_BROWS, _C), lambda i: (i, 0)),
            pl.BlockSpec((_BROWS,), lambda i: (i,)),
        ],
        out_specs=pl.BlockSpec((_WROWS, _CHUNK), lambda i: (i, 0)),
        out_shape=jax.ShapeDtypeStruct((_TROWS, _CHUNK), jnp.float32),
        compiler_params=pltpu.CompilerParams(
            dimension_semantics=("parallel",)),
    )(x, t)


def _tc_combine_body(at_ref, w_ref, o_ref):
    o_ref[0, 0] = -jnp.sum(at_ref[...] * w_ref[...]) * (1.0 / _N)


def _tc_combine(at, w):
    return pl.pallas_call(
        _tc_combine_body,
        out_specs=pl.BlockSpec(memory_space=pltpu.SMEM),
        out_shape=jax.ShapeDtypeStruct((1, 1), jnp.float32),
    )(at, w)


def kernel(inputs, targets, alpha, device=0):
    t = targets.astype(jnp.int32)
    a_flat = alpha.reshape(-1).astype(jnp.float32)
    at = _sc_alpha()(t.reshape(_TROWS, _CHUNK), a_flat)
    w = _tc_dense(inputs, t)
    loss = _tc_combine(at, w)
    return loss[0, 0]


# speedup vs baseline: 1.0281x; 1.0092x over previous
import functools

import jax
import jax.numpy as jnp
from jax import lax
from jax.experimental import pallas as pl
from jax.experimental.pallas import tpu as pltpu
from jax.experimental.pallas import tpu_sc as plsc

_N = 16384
_C = 1000

_NC = 2
_NS = 16
_NW = _NC * _NS
_RPW = _N // _NW
_CHUNK = 128
_NCH = _RPW // _CHUNK
_TROWS = _N // _CHUNK
_L = 16

_BROWS = 2048
_WROWS = _BROWS // _CHUNK


def _sc_alpha_body(t_hbm, a_hbm, at_hbm, t_v, at_v, sem):
    wid = lax.axis_index("s") * _NC + lax.axis_index("c")
    r0 = wid * _NCH
    pltpu.sync_copy(t_hbm.at[pl.ds(r0, _NCH)], t_v)
    copies = [
        pltpu.async_copy(a_hbm.at[t_v.at[ch]], at_v.at[ch], sem)
        for ch in range(_NCH)
    ]
    for cp in copies:
        cp.wait()
    pltpu.sync_copy(at_v, at_hbm.at[pl.ds(r0, _NCH)])


@functools.cache
def _sc_alpha():
    return functools.partial(
        pl.kernel,
        mesh=plsc.VectorSubcoreMesh(core_axis_name="c", subcore_axis_name="s"),
        out_type=jax.ShapeDtypeStruct((_TROWS, _CHUNK), jnp.float32),
        scratch_types=[
            pltpu.VMEM((_NCH, _CHUNK), jnp.int32),
            pltpu.VMEM((_NCH, _CHUNK), jnp.float32),
            pltpu.SemaphoreType.DMA,
        ],
    )(_sc_alpha_body)


def _tc_dense_body(x_ref, t_ref, w_ref):
    x = x_ref[...]
    cols = lax.broadcasted_iota(jnp.int32, (_BROWS, _C), 1)
    onehot = (cols == t_ref[...][:, None]).astype(jnp.float32)
    xt = jnp.sum(x * onehot, axis=1)
    m = jnp.max(x, axis=1)
    s = jnp.sum(jnp.exp(x - m[:, None]), axis=1)
    logp = xt - m - jnp.log(s)
    p = jnp.exp(logp)
    q = 1.0 - p
    w_ref[...] = (q * q * logp).reshape(_WROWS, _CHUNK)


def _tc_dense(x, t):
    return pl.pallas_call(
        _tc_dense_body,
        grid=(_N // _BROWS,),
        in_specs=[
            pl.BlockSpec((_BROWS, _C), lambda i: (i, 0)),
            pl.BlockSpec((_BROWS,), lambda i: (i,)),
        ],
        out_specs=pl.BlockSpec((_WROWS, _CHUNK), lambda i: (i, 0)),
        out_shape=jax.ShapeDtypeStruct((_TROWS, _CHUNK), jnp.float32),
        compiler_params=pltpu.CompilerParams(
            dimension_semantics=("parallel",)),
    )(x, t)


def _tc_combine_body(at_ref, w_ref, o_ref):
    o_ref[0, 0] = -jnp.sum(at_ref[...] * w_ref[...]) * (1.0 / _N)


def _tc_combine(at, w):
    return pl.pallas_call(
        _tc_combine_body,
        out_specs=pl.BlockSpec(memory_space=pltpu.SMEM),
        out_shape=jax.ShapeDtypeStruct((1, 1), jnp.float32),
    )(at, w)


def kernel(inputs, targets, alpha, device=0):
    t = targets.astype(jnp.int32)
    a_flat = alpha.reshape(-1).astype(jnp.float32)
    at = _sc_alpha()(t.reshape(_TROWS, _CHUNK), a_flat)
    w = _tc_dense(inputs, t)
    loss = _tc_combine(at, w)
    return loss[0, 0]


# speedup vs baseline: 1.0346x; 1.0063x over previous
import functools

import jax
import jax.numpy as jnp
from jax import lax
from jax.experimental import pallas as pl
from jax.experimental.pallas import tpu as pltpu
from jax.experimental.pallas import tpu_sc as plsc

_N = 16384
_C = 1000

_NC = 1
_NS = 16
_NW = _NC * _NS
_RPW = _N // _NW
_CHUNK = 128
_NCH = _RPW // _CHUNK
_TROWS = _N // _CHUNK

_BROWS = 2048
_WROWS = _BROWS // _CHUNK


def _sc_alpha_body(t_hbm, a_hbm, at_hbm, t_v, at_v, sem):
    wid = lax.axis_index("s") * _NC + lax.axis_index("c")
    r0 = wid * _NCH
    pltpu.sync_copy(t_hbm.at[pl.ds(r0, _NCH)], t_v)
    copies = [
        pltpu.async_copy(a_hbm.at[t_v.at[ch]], at_v.at[ch], sem)
        for ch in range(_NCH)
    ]
    for cp in copies:
        cp.wait()
    pltpu.sync_copy(at_v, at_hbm.at[pl.ds(r0, _NCH)])


@functools.cache
def _sc_alpha():
    return functools.partial(
        pl.kernel,
        mesh=plsc.VectorSubcoreMesh(core_axis_name="c", subcore_axis_name="s",
                                    num_cores=1),
        out_type=jax.ShapeDtypeStruct((_TROWS, _CHUNK), jnp.float32),
        scratch_types=[
            pltpu.VMEM((_NCH, _CHUNK), jnp.int32),
            pltpu.VMEM((_NCH, _CHUNK), jnp.float32),
            pltpu.SemaphoreType.DMA,
        ],
    )(_sc_alpha_body)


def _tc_dense_body(x_ref, t_ref, w_ref):
    x = x_ref[...]
    cols = lax.broadcasted_iota(jnp.int32, (_BROWS, _C), 1)
    onehot = (cols == t_ref[...][:, None]).astype(jnp.float32)
    xt = jnp.sum(x * onehot, axis=1)
    m = jnp.max(x, axis=1)
    s = jnp.sum(jnp.exp(x - m[:, None]), axis=1)
    logp = xt - m - jnp.log(s)
    p = jnp.exp(logp)
    q = 1.0 - p
    w_ref[...] = (q * q * logp).reshape(_WROWS, _CHUNK)


def _tc_dense(x, t):
    return pl.pallas_call(
        _tc_dense_body,
        grid=(_N // _BROWS,),
        in_specs=[
            pl.BlockSpec((_BROWS, _C), lambda i: (i, 0)),
            pl.BlockSpec((_BROWS,), lambda i: (i,)),
        ],
        out_specs=pl.BlockSpec((_WROWS, _CHUNK), lambda i: (i, 0)),
        out_shape=jax.ShapeDtypeStruct((_TROWS, _CHUNK), jnp.float32),
        compiler_params=pltpu.CompilerParams(
            dimension_semantics=("parallel",)),
    )(x, t)


def _tc_combine_body(at_ref, w_ref, o_ref):
    o_ref[0, 0] = -jnp.sum(at_ref[...] * w_ref[...]) * (1.0 / _N)


def _tc_combine(at, w):
    return pl.pallas_call(
        _tc_combine_body,
        out_specs=pl.BlockSpec(memory_space=pltpu.SMEM),
        out_shape=jax.ShapeDtypeStruct((1, 1), jnp.float32),
    )(at, w)


def kernel(inputs, targets, alpha, device=0):
    t = targets.astype(jnp.int32)
    a_flat = alpha.reshape(-1).astype(jnp.float32)
    at = _sc_alpha()(t.reshape(_TROWS, _CHUNK), a_flat)
    w = _tc_dense(inputs, t)
    loss = _tc_combine(at, w)
    return loss[0, 0]
